# Initial kernel scaffold; baseline (speedup 1.0000x reference)
import jax
import jax.numpy as jnp
from jax.experimental import pallas as pl


def kernel(x, table):
    raise NotImplementedError("write your pallas kernel here")


# speedup vs baseline: 1.4898x; 1.4898x over previous
import functools

import jax
import jax.numpy as jnp
from jax import lax
from jax.experimental import pallas as pl
from jax.experimental.pallas import tpu as pltpu
from jax.experimental.pallas import tpu_sc as plsc

_NC = 2
_NS = 16
_NW = _NC * _NS


@functools.lru_cache(maxsize=None)
def _build(n_rows: int, embed_dim: int):
  assert n_rows % _NW == 0
  b_per_w = n_rows // _NW
  chunk = 2560
  while b_per_w % chunk:
    chunk //= 2
  n_chunks = b_per_w // chunk

  mesh = plsc.VectorSubcoreMesh(core_axis_name="c", subcore_axis_name="s")

  @functools.partial(
      pl.kernel,
      mesh=mesh,
      out_type=jax.ShapeDtypeStruct((n_rows, embed_dim), jnp.float32),
      scratch_types=[
          pltpu.VMEM((chunk,), jnp.int32),
          pltpu.VMEM((chunk, embed_dim), jnp.float32),
          pltpu.SemaphoreType.DMA,
      ],
      compiler_params=pltpu.CompilerParams(use_tc_tiling_on_sc=False),
  )
  def gather_kernel(x_hbm, table_hbm, out_hbm, idx_v, rows_v, sem):
    wid = lax.axis_index("s") * _NC + lax.axis_index("c")
    base = wid * b_per_w

    def body(i, carry):
      off = base + i * chunk
      pltpu.sync_copy(x_hbm.at[pl.ds(off, chunk)], idx_v)
      pltpu.async_copy(table_hbm.at[idx_v], rows_v, sem).wait()
      pltpu.sync_copy(rows_v, out_hbm.at[pl.ds(off, chunk)])
      return carry

    lax.fori_loop(0, n_chunks, body, 0)

  return gather_kernel


def kernel(x, table):
  n_rows = x.size
  embed_dim = table.shape[1]
  out = _build(n_rows, embed_dim)(x.reshape(-1), table)
  return out.reshape(x.shape + (embed_dim,))


# speedup vs baseline: 1.4909x; 1.0007x over previous
import functools

import jax
import jax.numpy as jnp
from jax import lax
from jax.experimental import pallas as pl
from jax.experimental.pallas import tpu as pltpu
from jax.experimental.pallas import tpu_sc as plsc

_NC = 2
_NS = 16
_NW = _NC * _NS


@functools.lru_cache(maxsize=None)
def _build(n_rows: int, embed_dim: int):
  assert n_rows % _NW == 0
  b_per_w = n_rows // _NW
  chunk = 1600
  while b_per_w % chunk:
    chunk //= 2
  n_chunks = b_per_w // chunk

  mesh = plsc.VectorSubcoreMesh(core_axis_name="c", subcore_axis_name="s")

  @functools.partial(
      pl.kernel,
      mesh=mesh,
      out_type=jax.ShapeDtypeStruct((n_rows, embed_dim), jnp.float32),
      scratch_types=[
          pltpu.VMEM((chunk,), jnp.int32),
          pltpu.VMEM((chunk,), jnp.int32),
          pltpu.VMEM((chunk, embed_dim), jnp.float32),
          pltpu.VMEM((chunk, embed_dim), jnp.float32),
          pltpu.SemaphoreType.DMA,
          pltpu.SemaphoreType.DMA,
          pltpu.SemaphoreType.DMA,
          pltpu.SemaphoreType.DMA,
      ],
      compiler_params=pltpu.CompilerParams(use_tc_tiling_on_sc=False),
  )
  def gather_kernel(x_hbm, table_hbm, out_hbm,
                    idx0, idx1, rows0, rows1, g0, g1, o0, o1):
    wid = lax.axis_index("s") * _NC + lax.axis_index("c")
    base = wid * b_per_w

    idx = (idx0, idx1)
    rows = (rows0, rows1)
    gsem = (g0, g1)
    osem = (o0, o1)
    gathers = [None, None]
    stores = [None, None]

    for i in range(n_chunks):
      b = i & 1
      off = base + i * chunk
      if stores[b] is not None:
        stores[b].wait()
      pltpu.sync_copy(x_hbm.at[pl.ds(off, chunk)], idx[b])
      gathers[b] = pltpu.async_copy(table_hbm.at[idx[b]], rows[b], gsem[b])
      if i >= 1:
        nb = 1 - b
        gathers[nb].wait()
        stores[nb] = pltpu.async_copy(
            rows[nb], out_hbm.at[pl.ds(base + (i - 1) * chunk, chunk)],
            osem[nb])
    last = (n_chunks - 1) & 1
    gathers[last].wait()
    stores[last] = pltpu.async_copy(
        rows[last], out_hbm.at[pl.ds(base + (n_chunks - 1) * chunk, chunk)],
        osem[last])
    stores[0].wait()
    stores[1].wait()

  return gather_kernel


def kernel(x, table):
  n_rows = x.size
  embed_dim = table.shape[1]
  out = _build(n_rows, embed_dim)(x.reshape(-1), table)
  return out.reshape(x.shape + (embed_dim,))


# speedup vs baseline: 1.5005x; 1.0064x over previous
import functools

import jax
import jax.numpy as jnp
from jax import lax
from jax.experimental import pallas as pl
from jax.experimental.pallas import tpu as pltpu
from jax.experimental.pallas import tpu_sc as plsc

_NC = 2
_NS = 16
_NW = _NC * _NS


@functools.lru_cache(maxsize=None)
def _build(n_rows: int, embed_dim: int):
  assert n_rows % _NW == 0
  b_per_w = n_rows // _NW
  chunk = 1600
  while b_per_w % chunk:
    chunk //= 2
  n_chunks = b_per_w // chunk

  mesh = plsc.VectorSubcoreMesh(core_axis_name="c", subcore_axis_name="s")

  @functools.partial(
      pl.kernel,
      mesh=mesh,
      out_type=jax.ShapeDtypeStruct((n_rows, embed_dim), jnp.float32),
      scratch_types=[
          pltpu.VMEM((n_chunks, chunk), jnp.int32),
          pltpu.VMEM((chunk, embed_dim), jnp.float32),
          pltpu.VMEM((chunk, embed_dim), jnp.float32),
          pltpu.SemaphoreType.DMA,
          pltpu.SemaphoreType.DMA,
          pltpu.SemaphoreType.DMA,
          pltpu.SemaphoreType.DMA,
      ],
      compiler_params=pltpu.CompilerParams(use_tc_tiling_on_sc=False),
  )
  def gather_kernel(x_hbm, table_hbm, out_hbm,
                    idx_all, rows0, rows1, g0, g1, o0, o1):
    wid = lax.axis_index("s") * _NC + lax.axis_index("c")
    base = wid * b_per_w

    pltpu.sync_copy(x_hbm.at[wid], idx_all)

    rows = (rows0, rows1)
    gsem = (g0, g1)
    osem = (o0, o1)
    gathers = [None, None]
    stores = [None, None]

    for i in range(n_chunks):
      b = i & 1
      if stores[b] is not None:
        stores[b].wait()
      gathers[b] = pltpu.async_copy(table_hbm.at[idx_all.at[i]], rows[b],
                                    gsem[b])
      if i >= 1:
        nb = 1 - b
        gathers[nb].wait()
        stores[nb] = pltpu.async_copy(
            rows[nb], out_hbm.at[pl.ds(base + (i - 1) * chunk, chunk)],
            osem[nb])
    last = (n_chunks - 1) & 1
    gathers[last].wait()
    stores[last] = pltpu.async_copy(
        rows[last], out_hbm.at[pl.ds(base + (n_chunks - 1) * chunk, chunk)],
        osem[last])
    stores[0].wait()
    stores[1].wait()

  return gather_kernel, n_chunks, chunk


def kernel(x, table):
  n_rows = x.size
  embed_dim = table.shape[1]
  fn, n_chunks, chunk = _build(n_rows, embed_dim)
  x_tiled = x.reshape(_NW, n_chunks, chunk)
  out = fn(x_tiled, table)
  return out.reshape(x.shape + (embed_dim,))
